# Initial kernel scaffold; baseline (speedup 1.0000x reference)
import jax
import jax.numpy as jnp
from jax.experimental import pallas as pl


def kernel(inputs, inputs_ema, inputs_logits, inputs_logits_ema, features, labels, targets, indexes):
    raise NotImplementedError("write your pallas kernel here")


# speedup vs baseline: 19.4585x; 19.4585x over previous
import functools

import jax
import jax.numpy as jnp
from jax import lax
from jax.experimental import pallas as pl
from jax.experimental.pallas import tpu as pltpu
from jax.experimental.pallas import tpu_sc as plsc

_TEMP = 0.05
_K = 256
_EPS = 1e-6
_MASK_KEY = -1073741825
_CH = 128
_BIG = 1 << 20


def _keyify(s):
    b = lax.bitcast_convert_type(s, jnp.int32)
    return jnp.where(b >= 0, b, b ^ jnp.int32(0x7FFFFFFF))


def _unkey(k):
    b = jnp.where(k >= 0, k, k ^ jnp.int32(0x7FFFFFFF))
    return lax.bitcast_convert_type(b, jnp.float32)


def _s1_body(ROWS, NB, B, D, N):
    NJ = N // NB
    CPB = NB // _CH
    inv_t = 1.0 / _TEMP

    def body(x_ref, ema_ref, tgtc_ref, tgtr_ref, f_ref,
             keys_ref, pos_ref, mkey_ref, spos_ref, xn_ref, kms_ref):
        j = pl.program_id(0)

        @pl.when(j == 0)
        def _prep():
            x = x_ref[...]
            xn = x / (jnp.sqrt(jnp.sum(x * x, axis=1, keepdims=True)) + 1e-12)
            xn_ref[...] = xn
            ema = ema_ref[...]
            eman = ema / (jnp.sqrt(jnp.sum(ema * ema, axis=1, keepdims=True)) + 1e-12)
            bs = lax.dot_general(xn, eman, (((1,), (1,)), ((), ())),
                                 preferred_element_type=jnp.float32)
            bs = jnp.exp(bs * inv_t)
            pos = tgtc_ref[...] == tgtr_ref[...]
            spos_ref[...] = jnp.min(jnp.where(pos, bs, jnp.inf), axis=1, keepdims=True)

        xn = xn_ref[...]
        f = f_ref[...]
        s = lax.dot_general(xn, f, (((1,), (1,)), ((), ())),
                            preferred_element_type=jnp.float32)
        key = _keyify(s)
        col = lax.broadcasted_iota(jnp.int32, (ROWS, NB), 1) + j * NB
        t16 = tgtc_ref[...] * 16
        bmask = (col >= t16) & (col < t16 + 16)
        key = jnp.where(bmask, jnp.int32(_MASK_KEY), key)
        kparts = []
        for c in range(CPB):
            piece = key[:, c * _CH:(c + 1) * _CH]
            keys_ref[c] = piece
            kparts.append(jnp.max(piece, axis=1, keepdims=True))
        km = jnp.concatenate(kparts, axis=1)
        kms_ref[:, j] = km

        @pl.when(j == NJ - 1)
        def _search():
            kms = kms_ref[...]
            lo0 = jnp.full((ROWS, 1, 1), _MASK_KEY, jnp.int32)
            hi0 = jnp.max(jnp.max(kms, axis=2, keepdims=True), axis=1,
                          keepdims=True) + 1

            def sbody(_, carry):
                lo, hi = carry
                mid = lo + (hi - lo) // 2
                c = jnp.sum(jnp.sum((kms >= mid).astype(jnp.int32), axis=2,
                                    keepdims=True), axis=1, keepdims=True)
                ge = c >= _K
                return jnp.where(ge, mid, lo), jnp.where(ge, hi, mid)

            lo, _ = lax.fori_loop(0, 31, sbody, (lo0, hi0))
            mkey_ref[...] = lo.reshape(ROWS, 1)

            uJ = (lax.broadcasted_iota(jnp.int32, (NJ, NJ), 0)
                  < lax.broadcasted_iota(jnp.int32, (NJ, NJ), 1)).astype(jnp.float32)
            uC = (lax.broadcasted_iota(jnp.int32, (CPB, CPB), 0)
                  < lax.broadcasted_iota(jnp.int32, (CPB, CPB), 1)).astype(jnp.float32)

            def slots(mask_f):
                rsum = jnp.sum(mask_f, axis=2)
                prefj = lax.dot_general(rsum, uJ, (((1,), (0,)), ((), ())),
                                        preferred_element_type=jnp.float32)
                m2 = mask_f.reshape(ROWS * NJ, CPB)
                prefc = lax.dot_general(m2, uC, (((1,), (0,)), ((), ())),
                                        preferred_element_type=jnp.float32)
                pref = prefj.reshape(ROWS, NJ, 1) + prefc.reshape(ROWS, NJ, CPB)
                total = jnp.sum(jnp.sum(mask_f, axis=2, keepdims=True), axis=1,
                                keepdims=True)
                return pref, total

            gt = kms > lo
            eq = kms == lo
            pgt, cntgt = slots(gt.astype(jnp.float32))
            peq, _ = slots(eq.astype(jnp.float32))
            posP = jnp.where(gt, pgt.astype(jnp.int32),
                             jnp.where(eq, cntgt.astype(jnp.int32)
                                       + peq.astype(jnp.int32),
                                       jnp.int32(_BIG)))
            pos_ref[...] = posP

    return body


def _stage1(inputs, inputs_ema, tgtc, tgtr, features, B, D, N):
    ROWS = B
    NB = 2048 if N % 2048 == 0 else N
    NJ = N // NB
    CPB = NB // _CH
    NCH = N // _CH
    return pl.pallas_call(
        _s1_body(ROWS, NB, B, D, N),
        grid=(NJ,),
        in_specs=[
            pl.BlockSpec((B, D), lambda j: (0, 0)),
            pl.BlockSpec((B, D), lambda j: (0, 0)),
            pl.BlockSpec((B, 1), lambda j: (0, 0)),
            pl.BlockSpec((1, B), lambda j: (0, 0)),
            pl.BlockSpec((NB, D), lambda j: (j, 0)),
        ],
        out_specs=[
            pl.BlockSpec((CPB, B, _CH), lambda j: (j, 0, 0)),
            pl.BlockSpec((B, NJ, CPB), lambda j: (0, 0, 0)),
            pl.BlockSpec((B, 1), lambda j: (0, 0)),
            pl.BlockSpec((B, 1), lambda j: (0, 0)),
        ],
        out_shape=[
            jax.ShapeDtypeStruct((NCH, B, _CH), jnp.int32),
            jax.ShapeDtypeStruct((B, NJ, CPB), jnp.int32),
            jax.ShapeDtypeStruct((B, 1), jnp.int32),
            jax.ShapeDtypeStruct((B, 1), jnp.float32),
        ],
        scratch_shapes=[
            pltpu.VMEM((B, D), jnp.float32),
            pltpu.VMEM((B, NJ, CPB), jnp.int32),
        ],
    )(inputs, inputs_ema, tgtc, tgtr, features)


def _sc_compact_gather(posP, keys2d, B, NCH):
    mesh = plsc.VectorSubcoreMesh(core_axis_name="c", subcore_axis_name="s")
    NW = 32
    RPW = B // NW
    VPR = NCH // 16

    @functools.partial(
        pl.kernel, mesh=mesh,
        compiler_params=pltpu.CompilerParams(needs_layout_passes=False),
        out_type=jax.ShapeDtypeStruct((B * _K, _CH), jnp.int32),
        scratch_types=[
            pltpu.VMEM((VPR, 16), jnp.int32),
            pltpu.VMEM((128,), jnp.int32),
            pltpu.VMEM((128,), jnp.int32),
            pltpu.VMEM((_K, _CH), jnp.int32),
            pltpu.SemaphoreType.DMA,
        ],
    )
    def sc_k(pos_hbm, keys2_hbm, cand_hbm,
             posrow, idxv0, idxv1, candv, sem):
        cid = lax.axis_index("c")
        sid = lax.axis_index("s")
        wid = sid * 2 + cid
        iota16 = lax.iota(jnp.int32, 16)

        for k in range(RPW):
            r = wid * RPW + k
            pltpu.sync_copy(pos_hbm.at[pl.ds(r * VPR, VPR)], posrow)

            def p1(i2, _c):
                v = posrow[i2]
                ids = (iota16 + i2 * 16) * B + r
                m0 = v < 128
                m1 = (v >= 128) & (v < _K)
                plsc.store_scatter(idxv0, [v], ids, mask=m0)
                plsc.store_scatter(idxv1, [v - 128], ids, mask=m1)
                return _c

            lax.fori_loop(0, VPR, p1, jnp.int32(0))
            pltpu.async_copy(keys2_hbm.at[idxv0],
                             candv.at[pl.ds(0, 128)], sem).wait()
            pltpu.async_copy(keys2_hbm.at[idxv1],
                             candv.at[pl.ds(128, 128)], sem).wait()
            pltpu.sync_copy(candv, cand_hbm.at[pl.ds(r * _K, _K)])

    posf = posP.reshape(B * (NCH // 16), 16)
    return sc_k(posf, keys2d).reshape(B, _K, _CH)


def _s3_body(ROWS, B):
    inv_t = 1.0 / _TEMP

    def body(cand_ref, mkey_ref, spos_ref, out_ref):
        i = pl.program_id(0)
        cand = cand_ref[...]
        lo0 = mkey_ref[...].reshape(ROWS, 1, 1)
        hi0 = jnp.max(jnp.max(cand, axis=2, keepdims=True), axis=1,
                      keepdims=True) + 1

        def sbody(_, carry):
            lo, hi = carry
            mid = lo + (hi - lo) // 2
            c = jnp.sum(jnp.sum((cand >= mid).astype(jnp.int32), axis=2,
                                keepdims=True), axis=1, keepdims=True)
            ge = c >= _K
            return jnp.where(ge, mid, lo), jnp.where(ge, hi, mid)

        lo, _ = lax.fori_loop(0, 31, sbody, (lo0, hi0))
        vk = _unkey(lo)
        evk = jnp.exp(vk * inv_t)
        ge = cand >= lo
        e = jnp.exp(_unkey(cand) * inv_t)
        cnt = jnp.sum(jnp.sum(ge.astype(jnp.int32), axis=2, keepdims=True),
                      axis=1, keepdims=True)
        ssum = jnp.sum(jnp.sum(jnp.where(ge, e, 0.0), axis=2, keepdims=True),
                       axis=1, keepdims=True)
        sneg = (ssum - (cnt - _K).astype(jnp.float32) * evk).reshape(ROWS, 1)
        spos = spos_ref[...]
        frac = spos / (spos + sneg + _EPS)
        part = jnp.sum(-jnp.log(frac + 1e-6), axis=0, keepdims=True) / B

        @pl.when(i == 0)
        def _o0():
            out_ref[...] = part

        @pl.when(i > 0)
        def _on():
            out_ref[...] = out_ref[...] + part

    return body


def _stage3(cand, mkey, spos, B):
    ROWS = min(32, B)
    return pl.pallas_call(
        _s3_body(ROWS, B),
        grid=(B // ROWS,),
        in_specs=[
            pl.BlockSpec((ROWS, _K, _CH), lambda i: (i, 0, 0)),
            pl.BlockSpec((ROWS, 1), lambda i: (i, 0)),
            pl.BlockSpec((ROWS, 1), lambda i: (i, 0)),
        ],
        out_specs=pl.BlockSpec((1, 1), lambda i: (0, 0)),
        out_shape=jax.ShapeDtypeStruct((1, 1), jnp.float32),
    )(cand, mkey, spos)


def kernel(inputs, inputs_ema, inputs_logits, inputs_logits_ema, features,
           labels, targets, indexes):
    B, D = inputs.shape
    N = features.shape[0]
    NCH = N // _CH

    tgtc = targets.reshape(B, 1)
    tgtr = targets.reshape(1, B)

    keys3, posP, mkey, spos = _stage1(inputs, inputs_ema, tgtc, tgtr,
                                      features, B, D, N)
    cand = _sc_compact_gather(posP, keys3.reshape(NCH * B, _CH), B, NCH)
    out = _stage3(cand, mkey, spos, B)
    return out[0, 0]


# speedup vs baseline: 30.4528x; 1.5650x over previous
import functools

import jax
import jax.numpy as jnp
from jax import lax
from jax.experimental import pallas as pl
from jax.experimental.pallas import tpu as pltpu
from jax.experimental.pallas import tpu_sc as plsc

_TEMP = 0.05
_K = 256
_EPS = 1e-6
_MASK_KEY = -1073741825
_CH = 128
_BIG = 1 << 20


def _keyify(s):
    b = lax.bitcast_convert_type(s, jnp.int32)
    return jnp.where(b >= 0, b, b ^ jnp.int32(0x7FFFFFFF))


def _unkey(k):
    b = jnp.where(k >= 0, k, k ^ jnp.int32(0x7FFFFFFF))
    return lax.bitcast_convert_type(b, jnp.float32)


def _s1_body(ROWS, NB, B, D, N):
    NJ = N // NB
    CPB = NB // _CH
    inv_t = 1.0 / _TEMP

    def body(x_ref, ema_ref, tgtc_ref, tgtr_ref, f_ref,
             keys_ref, pos_ref, mkey_ref, spos_ref, xn_ref, kms_ref):
        j = pl.program_id(0)

        @pl.when(j == 0)
        def _prep():
            x = x_ref[...]
            xn = x / (jnp.sqrt(jnp.sum(x * x, axis=1, keepdims=True)) + 1e-12)
            xn_ref[...] = xn
            ema = ema_ref[...]
            eman = ema / (jnp.sqrt(jnp.sum(ema * ema, axis=1, keepdims=True)) + 1e-12)
            bs = lax.dot_general(xn, eman, (((1,), (1,)), ((), ())),
                                 preferred_element_type=jnp.float32)
            bs = jnp.exp(bs * inv_t)
            pos = tgtc_ref[...] == tgtr_ref[...]
            spos_ref[...] = jnp.min(jnp.where(pos, bs, jnp.inf), axis=1, keepdims=True)

        xn = xn_ref[...]
        f = f_ref[...]
        s = lax.dot_general(xn, f, (((1,), (1,)), ((), ())),
                            preferred_element_type=jnp.float32)
        key = _keyify(s)
        col = lax.broadcasted_iota(jnp.int32, (ROWS, NB), 1) + j * NB
        t16 = tgtc_ref[...] * 16
        bmask = (col >= t16) & (col < t16 + 16)
        key = jnp.where(bmask, jnp.int32(_MASK_KEY), key)
        kparts = []
        for c in range(CPB):
            piece = key[:, c * _CH:(c + 1) * _CH]
            keys_ref[c] = piece
            kparts.append(jnp.max(piece, axis=1, keepdims=True))
        km = jnp.concatenate(kparts, axis=1)
        kms_ref[:, j] = km

        @pl.when(j == NJ - 1)
        def _search():
            kms = kms_ref[...]
            lo0 = jnp.full((ROWS, 1, 1), _MASK_KEY, jnp.int32)
            hi0 = jnp.max(jnp.max(kms, axis=2, keepdims=True), axis=1,
                          keepdims=True) + 1

            def sbody(_, carry):
                lo, hi = carry
                mid = lo + (hi - lo) // 2
                c = jnp.sum(jnp.sum((kms >= mid).astype(jnp.int32), axis=2,
                                    keepdims=True), axis=1, keepdims=True)
                ge = c >= _K
                return jnp.where(ge, mid, lo), jnp.where(ge, hi, mid)

            lo, _ = lax.fori_loop(0, 31, sbody, (lo0, hi0))
            mkey_ref[...] = lo.reshape(ROWS, 1)

            uJ = (lax.broadcasted_iota(jnp.int32, (NJ, NJ), 0)
                  < lax.broadcasted_iota(jnp.int32, (NJ, NJ), 1)).astype(jnp.float32)
            uC = (lax.broadcasted_iota(jnp.int32, (CPB, CPB), 0)
                  < lax.broadcasted_iota(jnp.int32, (CPB, CPB), 1)).astype(jnp.float32)

            def slots(mask_f):
                rsum = jnp.sum(mask_f, axis=2)
                prefj = lax.dot_general(rsum, uJ, (((1,), (0,)), ((), ())),
                                        preferred_element_type=jnp.float32)
                m2 = mask_f.reshape(ROWS * NJ, CPB)
                prefc = lax.dot_general(m2, uC, (((1,), (0,)), ((), ())),
                                        preferred_element_type=jnp.float32)
                pref = prefj.reshape(ROWS, NJ, 1) + prefc.reshape(ROWS, NJ, CPB)
                total = jnp.sum(jnp.sum(mask_f, axis=2, keepdims=True), axis=1,
                                keepdims=True)
                return pref, total

            gt = kms > lo
            eq = kms == lo
            pgt, cntgt = slots(gt.astype(jnp.float32))
            peq, _ = slots(eq.astype(jnp.float32))
            posP = jnp.where(gt, pgt.astype(jnp.int32),
                             jnp.where(eq, cntgt.astype(jnp.int32)
                                       + peq.astype(jnp.int32),
                                       jnp.int32(_BIG)))
            pos_ref[...] = posP

    return body


def _stage1(inputs, inputs_ema, tgtc, tgtr, features, B, D, N):
    ROWS = B
    NB = 2048 if N % 2048 == 0 else N
    NJ = N // NB
    CPB = NB // _CH
    NCH = N // _CH
    return pl.pallas_call(
        _s1_body(ROWS, NB, B, D, N),
        grid=(NJ,),
        in_specs=[
            pl.BlockSpec((B, D), lambda j: (0, 0)),
            pl.BlockSpec((B, D), lambda j: (0, 0)),
            pl.BlockSpec((B, 1), lambda j: (0, 0)),
            pl.BlockSpec((1, B), lambda j: (0, 0)),
            pl.BlockSpec((NB, D), lambda j: (j, 0)),
        ],
        out_specs=[
            pl.BlockSpec((CPB, B, _CH), lambda j: (j, 0, 0)),
            pl.BlockSpec((B, NJ, CPB), lambda j: (0, 0, 0)),
            pl.BlockSpec((B, 1), lambda j: (0, 0)),
            pl.BlockSpec((B, 1), lambda j: (0, 0)),
        ],
        out_shape=[
            jax.ShapeDtypeStruct((NCH, B, _CH), jnp.int32),
            jax.ShapeDtypeStruct((B, NJ, CPB), jnp.int32),
            jax.ShapeDtypeStruct((B, 1), jnp.int32),
            jax.ShapeDtypeStruct((B, 1), jnp.float32),
        ],
        scratch_shapes=[
            pltpu.VMEM((B, D), jnp.float32),
            pltpu.VMEM((B, NJ, CPB), jnp.int32),
        ],
    )(inputs, inputs_ema, tgtc, tgtr, features)


def _sc_compact_gather(posP, keys2d, B, NCH):
    mesh = plsc.VectorSubcoreMesh(core_axis_name="c", subcore_axis_name="s")
    NW = 32
    RPW = B // NW
    VPR = NCH // 16

    @functools.partial(
        pl.kernel, mesh=mesh,
        compiler_params=pltpu.CompilerParams(needs_layout_passes=False),
        out_type=jax.ShapeDtypeStruct((B * _K, _CH), jnp.int32),
        scratch_types=[
            pltpu.VMEM((VPR, 16), jnp.int32),
            pltpu.VMEM((128,), jnp.int32),
            pltpu.VMEM((128,), jnp.int32),
            pltpu.VMEM((_K, _CH), jnp.int32),
            pltpu.SemaphoreType.DMA,
        ],
    )
    def sc_k(pos_hbm, keys2_hbm, cand_hbm,
             posrow, idxv0, idxv1, candv, sem):
        cid = lax.axis_index("c")
        sid = lax.axis_index("s")
        wid = sid * 2 + cid
        iota16 = lax.iota(jnp.int32, 16)

        for k in range(RPW):
            r = wid * RPW + k
            pltpu.sync_copy(pos_hbm.at[pl.ds(r * VPR, VPR)], posrow)

            def p1(i2, _c):
                v = posrow[i2]
                ids = (iota16 + i2 * 16) * B + r
                m0 = v < 128
                m1 = (v >= 128) & (v < _K)
                plsc.store_scatter(idxv0, [v], ids, mask=m0)
                plsc.store_scatter(idxv1, [v - 128], ids, mask=m1)
                return _c

            lax.fori_loop(0, VPR, p1, jnp.int32(0))
            pltpu.async_copy(keys2_hbm.at[idxv0],
                             candv.at[pl.ds(0, 128)], sem).wait()
            pltpu.async_copy(keys2_hbm.at[idxv1],
                             candv.at[pl.ds(128, 128)], sem).wait()
            pltpu.sync_copy(candv, cand_hbm.at[pl.ds(r * _K, _K)])

    posf = posP.reshape(B * (NCH // 16), 16)
    return sc_k(posf, keys2d).reshape(B, _K, _CH)


def _s3_body(ROWS, B):
    inv_t = 1.0 / _TEMP

    def body(cand_ref, mkey_ref, spos_ref, out_ref):
        i = pl.program_id(0)
        cand = cand_ref[...].reshape(ROWS, _K * _CH)
        ones = jnp.ones((_K * _CH, 1), jnp.float32)
        lo0 = mkey_ref[...]
        hi0 = jnp.max(cand, axis=1, keepdims=True) + 1

        def count_ge(th):
            gef = (cand >= th).astype(jnp.float32)
            return lax.dot_general(gef, ones, (((1,), (0,)), ((), ())),
                                   preferred_element_type=jnp.float32)

        def scond(carry):
            lo, hi = carry
            return jnp.max(hi - lo) > 1

        def sbody(carry):
            lo, hi = carry
            mid = lo + (hi - lo) // 2
            ge = count_ge(mid) >= _K
            return jnp.where(ge, mid, lo), jnp.where(ge, hi, mid)

        lo, _ = lax.while_loop(scond, sbody, (lo0, hi0))
        vk = _unkey(lo)
        evk = jnp.exp(vk * inv_t)
        ge = cand >= lo
        e = jnp.where(ge, jnp.exp(_unkey(cand) * inv_t), 0.0)
        both = jnp.concatenate([ge.astype(jnp.float32), e], axis=0)
        red = lax.dot_general(both, ones, (((1,), (0,)), ((), ())),
                              preferred_element_type=jnp.float32)
        cnt = red[:ROWS]
        ssum = red[ROWS:]
        sneg = ssum - (cnt - _K) * evk
        spos = spos_ref[...]
        frac = spos / (spos + sneg + _EPS)
        part = jnp.sum(-jnp.log(frac + 1e-6), axis=0, keepdims=True) / B

        @pl.when(i == 0)
        def _o0():
            out_ref[...] = part

        @pl.when(i > 0)
        def _on():
            out_ref[...] = out_ref[...] + part

    return body


def _stage3(cand, mkey, spos, B):
    ROWS = min(32, B)
    return pl.pallas_call(
        _s3_body(ROWS, B),
        grid=(B // ROWS,),
        in_specs=[
            pl.BlockSpec((ROWS, _K, _CH), lambda i: (i, 0, 0)),
            pl.BlockSpec((ROWS, 1), lambda i: (i, 0)),
            pl.BlockSpec((ROWS, 1), lambda i: (i, 0)),
        ],
        out_specs=pl.BlockSpec((1, 1), lambda i: (0, 0)),
        out_shape=jax.ShapeDtypeStruct((1, 1), jnp.float32),
    )(cand, mkey, spos)


def kernel(inputs, inputs_ema, inputs_logits, inputs_logits_ema, features,
           labels, targets, indexes):
    B, D = inputs.shape
    N = features.shape[0]
    NCH = N // _CH

    tgtc = targets.reshape(B, 1)
    tgtr = targets.reshape(1, B)

    keys3, posP, mkey, spos = _stage1(inputs, inputs_ema, tgtc, tgtr,
                                      features, B, D, N)
    cand = _sc_compact_gather(posP, keys3.reshape(NCH * B, _CH), B, NCH)
    out = _stage3(cand, mkey, spos, B)
    return out[0, 0]


# speedup vs baseline: 38.7390x; 1.2721x over previous
import functools

import jax
import jax.numpy as jnp
from jax import lax
from jax.experimental import pallas as pl
from jax.experimental.pallas import tpu as pltpu
from jax.experimental.pallas import tpu_sc as plsc

_TEMP = 0.05
_K = 256
_EPS = 1e-6
_MASK_KEY = -1073741825
_CH = 128
_BIG = 1 << 20


def _keyify(s):
    b = lax.bitcast_convert_type(s, jnp.int32)
    return jnp.where(b >= 0, b, b ^ jnp.int32(0x7FFFFFFF))


def _unkey(k):
    b = jnp.where(k >= 0, k, k ^ jnp.int32(0x7FFFFFFF))
    return lax.bitcast_convert_type(b, jnp.float32)


def _s1_body(ROWS, NB, B, D, N):
    NJ = N // NB
    CPB = NB // _CH
    inv_t = 1.0 / _TEMP

    def body(x_ref, ema_ref, tgtc_ref, tgtr_ref, f_ref,
             keys_ref, pos_ref, mkey_ref, spos_ref, xn_ref, kms_ref):
        j = pl.program_id(0)

        @pl.when(j == 0)
        def _prep():
            x = x_ref[...]
            xn = x / (jnp.sqrt(jnp.sum(x * x, axis=1, keepdims=True)) + 1e-12)
            xn_ref[...] = xn
            ema = ema_ref[...]
            eman = ema / (jnp.sqrt(jnp.sum(ema * ema, axis=1, keepdims=True)) + 1e-12)
            bs = lax.dot_general(xn, eman, (((1,), (1,)), ((), ())),
                                 preferred_element_type=jnp.float32)
            bs = jnp.exp(bs * inv_t)
            pos = tgtc_ref[...] == tgtr_ref[...]
            spos_ref[...] = jnp.min(jnp.where(pos, bs, jnp.inf), axis=1, keepdims=True)

        xn = xn_ref[...]
        f = f_ref[...]
        s = lax.dot_general(xn, f, (((1,), (1,)), ((), ())),
                            preferred_element_type=jnp.float32)
        key = _keyify(s)
        col = lax.broadcasted_iota(jnp.int32, (ROWS, NB), 1) + j * NB
        t16 = tgtc_ref[...] * 16
        bmask = (col >= t16) & (col < t16 + 16)
        key = jnp.where(bmask, jnp.int32(_MASK_KEY), key)
        kparts = []
        for c in range(CPB):
            piece = key[:, c * _CH:(c + 1) * _CH]
            keys_ref[c] = piece
            kparts.append(jnp.max(piece, axis=1, keepdims=True))
        km = jnp.concatenate(kparts, axis=1)
        kms_ref[:, j] = km

        @pl.when(j == NJ - 1)
        def _search():
            NCH = NJ * CPB
            kms = kms_ref[...].reshape(ROWS, NCH)
            ones = jnp.ones((NCH, 1), jnp.float32)
            lo0 = jnp.min(kms, axis=1, keepdims=True)
            hi0 = jnp.max(kms, axis=1, keepdims=True) + 1

            def count_ge(th):
                gef = (kms >= th).astype(jnp.float32)
                return lax.dot_general(gef, ones, (((1,), (0,)), ((), ())),
                                       preferred_element_type=jnp.float32)

            def scond(carry):
                lo, hi = carry
                return jnp.max(hi - lo) > 1

            def sbody(carry):
                lo, hi = carry
                mid = lo + (hi - lo) // 2
                ge = count_ge(mid) >= _K
                return jnp.where(ge, mid, lo), jnp.where(ge, hi, mid)

            lo, _ = lax.while_loop(scond, sbody, (lo0, hi0))
            mkey_ref[...] = lo

            uT = (lax.broadcasted_iota(jnp.int32, (NCH, NCH), 0)
                  < lax.broadcasted_iota(jnp.int32, (NCH, NCH), 1)).astype(jnp.float32)
            gt = kms > lo
            eq = kms == lo
            gtf = gt.astype(jnp.float32)
            eqf = eq.astype(jnp.float32)
            both = jnp.concatenate([gtf, eqf], axis=0)
            pref = lax.dot_general(both, uT, (((1,), (0,)), ((), ())),
                                   preferred_element_type=jnp.float32)
            cntgt = lax.dot_general(gtf, ones, (((1,), (0,)), ((), ())),
                                    preferred_element_type=jnp.float32)
            posP = jnp.where(gt, pref[:ROWS].astype(jnp.int32),
                             jnp.where(eq, cntgt.astype(jnp.int32)
                                       + pref[ROWS:].astype(jnp.int32),
                                       jnp.int32(_BIG)))
            pos_ref[...] = posP

    return body


def _stage1(inputs, inputs_ema, tgtc, tgtr, features, B, D, N):
    ROWS = B
    NB = 2048 if N % 2048 == 0 else N
    NJ = N // NB
    CPB = NB // _CH
    NCH = N // _CH
    return pl.pallas_call(
        _s1_body(ROWS, NB, B, D, N),
        grid=(NJ,),
        in_specs=[
            pl.BlockSpec((B, D), lambda j: (0, 0)),
            pl.BlockSpec((B, D), lambda j: (0, 0)),
            pl.BlockSpec((B, 1), lambda j: (0, 0)),
            pl.BlockSpec((1, B), lambda j: (0, 0)),
            pl.BlockSpec((NB, D), lambda j: (j, 0)),
        ],
        out_specs=[
            pl.BlockSpec((CPB, B, _CH), lambda j: (j, 0, 0)),
            pl.BlockSpec((B, NCH), lambda j: (0, 0)),
            pl.BlockSpec((B, 1), lambda j: (0, 0)),
            pl.BlockSpec((B, 1), lambda j: (0, 0)),
        ],
        out_shape=[
            jax.ShapeDtypeStruct((NCH, B, _CH), jnp.int32),
            jax.ShapeDtypeStruct((B, NCH), jnp.int32),
            jax.ShapeDtypeStruct((B, 1), jnp.int32),
            jax.ShapeDtypeStruct((B, 1), jnp.float32),
        ],
        scratch_shapes=[
            pltpu.VMEM((B, D), jnp.float32),
            pltpu.VMEM((B, NJ, CPB), jnp.int32),
        ],
    )(inputs, inputs_ema, tgtc, tgtr, features)


def _sc_compact_gather(posP, keys2d, B, NCH):
    mesh = plsc.VectorSubcoreMesh(core_axis_name="c", subcore_axis_name="s")
    NW = 32
    RPW = B // NW
    VPR = NCH // 16

    @functools.partial(
        pl.kernel, mesh=mesh,
        compiler_params=pltpu.CompilerParams(needs_layout_passes=False),
        out_type=jax.ShapeDtypeStruct((B * _K, _CH), jnp.int32),
        scratch_types=[
            pltpu.VMEM((NCH,), jnp.int32),
            pltpu.VMEM((128,), jnp.int32),
            pltpu.VMEM((128,), jnp.int32),
            pltpu.VMEM((_K, _CH), jnp.int32),
            pltpu.SemaphoreType.DMA,
        ],
    )
    def sc_k(pos_hbm, keys2_hbm, cand_hbm,
             posrow, idxv0, idxv1, candv, sem):
        cid = lax.axis_index("c")
        sid = lax.axis_index("s")
        wid = sid * 2 + cid
        iota16 = lax.iota(jnp.int32, 16)

        for k in range(RPW):
            r = wid * RPW + k
            pltpu.sync_copy(pos_hbm.at[pl.ds(r * NCH, NCH)], posrow)

            def p1(i2, _c):
                v = posrow[pl.ds(i2 * 16, 16)]
                ids = (iota16 + i2 * 16) * B + r
                m0 = v < 128
                m1 = (v >= 128) & (v < _K)
                plsc.store_scatter(idxv0, [v], ids, mask=m0)
                plsc.store_scatter(idxv1, [v - 128], ids, mask=m1)
                return _c

            lax.fori_loop(0, VPR, p1, jnp.int32(0))
            pltpu.async_copy(keys2_hbm.at[idxv0],
                             candv.at[pl.ds(0, 128)], sem).wait()
            pltpu.async_copy(keys2_hbm.at[idxv1],
                             candv.at[pl.ds(128, 128)], sem).wait()
            pltpu.sync_copy(candv, cand_hbm.at[pl.ds(r * _K, _K)])

    posf = posP.reshape(B * NCH)
    return sc_k(posf, keys2d).reshape(B, _K, _CH)


def _s3_body(ROWS, B):
    inv_t = 1.0 / _TEMP

    def body(cand_ref, mkey_ref, spos_ref, out_ref):
        i = pl.program_id(0)
        cand = cand_ref[...].reshape(ROWS, _K * _CH)
        ones = jnp.ones((_K * _CH, 1), jnp.float32)
        lo0 = mkey_ref[...]
        hi0 = jnp.max(cand, axis=1, keepdims=True) + 1

        def count_ge(th):
            gef = (cand >= th).astype(jnp.float32)
            return lax.dot_general(gef, ones, (((1,), (0,)), ((), ())),
                                   preferred_element_type=jnp.float32)

        def scond(carry):
            lo, hi = carry
            return jnp.max(hi - lo) > 1

        def sbody(carry):
            lo, hi = carry
            mid = lo + (hi - lo) // 2
            ge = count_ge(mid) >= _K
            return jnp.where(ge, mid, lo), jnp.where(ge, hi, mid)

        lo, _ = lax.while_loop(scond, sbody, (lo0, hi0))
        vk = _unkey(lo)
        evk = jnp.exp(vk * inv_t)
        ge = cand >= lo
        e = jnp.where(ge, jnp.exp(_unkey(cand) * inv_t), 0.0)
        both = jnp.concatenate([ge.astype(jnp.float32), e], axis=0)
        red = lax.dot_general(both, ones, (((1,), (0,)), ((), ())),
                              preferred_element_type=jnp.float32)
        cnt = red[:ROWS]
        ssum = red[ROWS:]
        sneg = ssum - (cnt - _K) * evk
        spos = spos_ref[...]
        frac = spos / (spos + sneg + _EPS)
        part = jnp.sum(-jnp.log(frac + 1e-6), axis=0, keepdims=True) / B

        @pl.when(i == 0)
        def _o0():
            out_ref[...] = part

        @pl.when(i > 0)
        def _on():
            out_ref[...] = out_ref[...] + part

    return body


def _stage3(cand, mkey, spos, B):
    ROWS = min(32, B)
    return pl.pallas_call(
        _s3_body(ROWS, B),
        grid=(B // ROWS,),
        in_specs=[
            pl.BlockSpec((ROWS, _K, _CH), lambda i: (i, 0, 0)),
            pl.BlockSpec((ROWS, 1), lambda i: (i, 0)),
            pl.BlockSpec((ROWS, 1), lambda i: (i, 0)),
        ],
        out_specs=pl.BlockSpec((1, 1), lambda i: (0, 0)),
        out_shape=jax.ShapeDtypeStruct((1, 1), jnp.float32),
    )(cand, mkey, spos)


def kernel(inputs, inputs_ema, inputs_logits, inputs_logits_ema, features,
           labels, targets, indexes):
    B, D = inputs.shape
    N = features.shape[0]
    NCH = N // _CH

    tgtc = targets.reshape(B, 1)
    tgtr = targets.reshape(1, B)

    keys3, posP, mkey, spos = _stage1(inputs, inputs_ema, tgtc, tgtr,
                                      features, B, D, N)
    cand = _sc_compact_gather(posP, keys3.reshape(NCH * B, _CH), B, NCH)
    out = _stage3(cand, mkey, spos, B)
    return out[0, 0]
